# Initial kernel scaffold; baseline (speedup 1.0000x reference)
import jax
import jax.numpy as jnp
from jax.experimental import pallas as pl


def kernel(query, cases, case_labels):
    raise NotImplementedError("write your pallas kernel here")


# speedup vs baseline: 1.9096x; 1.9096x over previous
import functools

import jax
import jax.numpy as jnp
from jax import lax
from jax.experimental import pallas as pl
from jax.experimental.pallas import tpu as pltpu
from jax.experimental.pallas import tpu_sc as plsc

B = 32
N = 100000
F = 16
TOPK = 32
LANES = 128
GROUPS = LANES // F
NR = N // GROUPS
RB = 64
CBLK = 2048

SCCHUNK = 20000
NCHUNK = N // SCCHUNK
VPC = SCCHUNK // 16
NEG = float("-inf")


def _fa_ca_body(qpat_ref, cases_ref, fa_ref, ca_ref):
    qp = qpat_ref[...]
    cs = cases_ref[...]
    fa = jnp.exp(-jnp.abs(qp[:, None, :] - cs[None, :, :]))
    fa_ref[...] = fa
    r = lax.broadcasted_iota(jnp.int32, (LANES, GROUPS), 0)
    c = lax.broadcasted_iota(jnp.int32, (LANES, GROUPS), 1)
    g = (r // F == c).astype(jnp.float32)
    e = fa.reshape(B * RB, LANES)
    s = jnp.dot(e, g, preferred_element_type=jnp.float32,
                precision=lax.Precision.HIGHEST)
    ca_ref[...] = s.reshape(B, RB, GROUPS) * (1.0 / F)


def _fa_ca(qpat, cases3):
    grid = (NR + RB - 1) // RB
    return pl.pallas_call(
        _fa_ca_body,
        grid=(grid,),
        in_specs=[
            pl.BlockSpec((B, LANES), lambda i: (0, 0)),
            pl.BlockSpec((RB, LANES), lambda i: (i, 0)),
        ],
        out_specs=[
            pl.BlockSpec((B, RB, LANES), lambda i: (0, i, 0)),
            pl.BlockSpec((B, RB, GROUPS), lambda i: (0, i, 0)),
        ],
        out_shape=[
            jax.ShapeDtypeStruct((B, NR, LANES), jnp.float32),
            jax.ShapeDtypeStruct((B, NR, GROUPS), jnp.float32),
        ],
    )(qpat, cases3)


def _merge(tk0, tv0, tk1, tv1, ck, cv):
    ck, cv = plsc.sort_key_val(ck, cv, descending=True)
    rk = lax.rev(tk1, (0,))
    rv = lax.rev(tv1, (0,))
    m1 = ck > rk
    hk = jnp.where(m1, ck, rk)
    hv = jnp.where(m1, cv, rv)
    hk, hv = plsc.sort_key_val(hk, hv, descending=True)
    rhk = lax.rev(hk, (0,))
    rhv = lax.rev(hv, (0,))
    m2 = tk0 >= rhk
    ak = jnp.where(m2, tk0, rhk)
    av = jnp.where(m2, tv0, rhv)
    bk = jnp.where(m2, rhk, tk0)
    bv = jnp.where(m2, rhv, tv0)
    nk0, nv0 = plsc.sort_key_val(ak, av, descending=True)
    nk1, nv1 = plsc.sort_key_val(bk, bv, descending=True)
    return nk0, nv0, nk1, nv1


def _topk_kernel_body(ca_hbm, lab_hbm, out_hbm, buf, idxbuf, labbuf, outbuf,
                      gsem):
    c = lax.axis_index("c")
    s = lax.axis_index("s")
    row = s * 2 + c

    lane = lax.iota(jnp.int32, 16)
    t0k = jnp.full((16,), NEG, jnp.float32)
    t0v = jnp.zeros((16,), jnp.int32)
    t1k = jnp.full((16,), NEG, jnp.float32)
    t1v = jnp.zeros((16,), jnp.int32)
    thresh = jnp.float32(NEG)

    def chunk_body(k, carry):
        t0k, t0v, t1k, t1v, thresh = carry
        pltpu.sync_copy(ca_hbm.at[pl.ds(row * N + k * SCCHUNK, SCCHUNK)], buf)
        base0 = k * SCCHUNK

        def vec_body(i, carry):
            t0k, t0v, t1k, t1v, thresh = carry
            v = buf[pl.ds(i * 16, 16)]
            m = v > thresh

            def do_merge(t0k, t0v, t1k, t1v):
                ck = jnp.where(m, v, NEG)
                cv = base0 + i * 16 + lane
                t0k, t0v, t1k, t1v = _merge(t0k, t0v, t1k, t1v, ck, cv)
                return t0k, t0v, t1k, t1v, jnp.min(t1k)

            def no_merge(t0k, t0v, t1k, t1v):
                return t0k, t0v, t1k, t1v, thresh

            return lax.cond(jnp.any(m), do_merge, no_merge, t0k, t0v, t1k, t1v)

        return lax.fori_loop(0, VPC, vec_body, (t0k, t0v, t1k, t1v, thresh))

    t0k, t0v, t1k, t1v, thresh = lax.fori_loop(
        0, NCHUNK, chunk_body, (t0k, t0v, t1k, t1v, thresh))

    idxbuf[pl.ds(0, 16)] = t0v
    idxbuf[pl.ds(16, 16)] = t1v
    pltpu.async_copy(lab_hbm.at[idxbuf], labbuf, gsem).wait()
    l0 = labbuf[pl.ds(0, 16)]
    l1 = labbuf[pl.ds(16, 16)]
    kth = jnp.min(t1k)
    denom = jnp.sum(t0k) + jnp.sum(t1k)
    sumlab = jnp.sum(l0) + jnp.sum(l1)
    dot = jnp.sum(t0k * l0) + jnp.sum(t1k * l1)
    recipv = 1.0 / (jnp.broadcast_to(denom, (16,)) + 1e-10)
    predv = jnp.broadcast_to(dot, (16,)) * recipv
    outv = jnp.where(lane == 0, jnp.broadcast_to(kth, (16,)),
           jnp.where(lane == 1, jnp.broadcast_to(sumlab * (1.0 / 32.0), (16,)),
           jnp.where(lane == 2, predv,
           jnp.where(lane == 3, recipv, 0.0))))
    outbuf[...] = outv
    pltpu.sync_copy(outbuf, out_hbm.at[pl.ds(row * 16, 16)])


def _topk_stats(ca_flat, labels):
    mesh = plsc.VectorSubcoreMesh(core_axis_name="c", subcore_axis_name="s")
    f = pl.kernel(
        _topk_kernel_body,
        out_type=jax.ShapeDtypeStruct((B * 16,), jnp.float32),
        mesh=mesh,
        scratch_types=[
            pltpu.VMEM((SCCHUNK,), jnp.float32),
            pltpu.VMEM((2 * 16,), jnp.int32),
            pltpu.VMEM((2 * 16,), jnp.float32),
            pltpu.VMEM((16,), jnp.float32),
            pltpu.SemaphoreType.DMA,
        ],
        compiler_params=pltpu.CompilerParams(needs_layout_passes=False),
    )
    return f(ca_flat, labels).reshape(B, 16)


def _sm_body(ca_ref, thr_ref, rec_ref, sm_ref):
    ca = ca_ref[...]
    thr = thr_ref[...]
    rec = rec_ref[...]
    sm_ref[...] = jnp.where(ca >= thr, ca, 0.0) * rec


def _sm(ca, thr, rec):
    grid = (N + CBLK - 1) // CBLK
    return pl.pallas_call(
        _sm_body,
        grid=(grid,),
        in_specs=[
            pl.BlockSpec((B, CBLK), lambda i: (0, i)),
            pl.BlockSpec((B, 1), lambda i: (0, 0)),
            pl.BlockSpec((B, 1), lambda i: (0, 0)),
        ],
        out_specs=pl.BlockSpec((B, CBLK), lambda i: (0, i)),
        out_shape=jax.ShapeDtypeStruct((B, N), jnp.float32),
    )(ca, thr, rec)


def kernel(query, cases, case_labels):
    qpat = jnp.tile(query, (1, GROUPS))
    cases3 = cases.reshape(NR, LANES)
    fa3, ca3 = _fa_ca(qpat, cases3)
    fa = fa3.reshape(B, N, F)
    ca = ca3.reshape(B, N)

    stats = _topk_stats(ca.reshape(-1), case_labels)
    kth = stats[:, 0:1]
    output = stats[:, 1]
    predicted = stats[:, 2]
    recip = stats[:, 3:4]

    sm = _sm(ca, kth, recip)
    return (fa, sm, output, predicted)


# speedup vs baseline: 2.1489x; 1.1253x over previous
import functools

import jax
import jax.numpy as jnp
from jax import lax
from jax.experimental import pallas as pl
from jax.experimental.pallas import tpu as pltpu
from jax.experimental.pallas import tpu_sc as plsc

B = 32
N = 100000
NP = 100352
F = 16
TOPK = 32
LANES = 128
GROUPS = LANES // F
NR = N // GROUPS
RB = 64
CBLK = 2048

SCCHUNK = NP // 8
VPC = SCCHUNK // 16
NEG = float("-inf")


def _fa_ca_body(qpat_ref, cases_ref, ct_ref, fa_ref, ca_ref):
    qp = qpat_ref[...]
    cs = cases_ref[...]
    fa = jnp.exp(-jnp.abs(qp[:, None, :] - cs[None, :, :]))
    fa_ref[...] = fa
    acc = jnp.zeros((B, RB * GROUPS), jnp.float32)
    for f in range(F):
        qf = qpat_ref[:, f:f + 1]
        cf = ct_ref[f:f + 1, :]
        acc = acc + jnp.exp(-jnp.abs(qf - cf))
    ca = acc * (1.0 / F)
    i = pl.program_id(0)
    gcol = i * (RB * GROUPS) + lax.broadcasted_iota(
        jnp.int32, (B, RB * GROUPS), 1)
    ca_ref[...] = jnp.where(gcol < N, ca, 0.0)


def _fa_ca(qpat, cases3, casesT):
    grid = (NR + RB - 1) // RB
    return pl.pallas_call(
        _fa_ca_body,
        grid=(grid,),
        in_specs=[
            pl.BlockSpec((B, LANES), lambda i: (0, 0)),
            pl.BlockSpec((RB, LANES), lambda i: (i, 0)),
            pl.BlockSpec((F, RB * GROUPS), lambda i: (0, i)),
        ],
        out_specs=[
            pl.BlockSpec((B, RB, LANES), lambda i: (0, i, 0)),
            pl.BlockSpec((B, RB * GROUPS), lambda i: (0, i)),
        ],
        out_shape=[
            jax.ShapeDtypeStruct((B, NR, LANES), jnp.float32),
            jax.ShapeDtypeStruct((B, NP), jnp.float32),
        ],
    )(qpat, cases3, casesT)


def _merge(tk0, tv0, tk1, tv1, ck, cv):
    ck, cv = plsc.sort_key_val(ck, cv, descending=True)
    rk = lax.rev(tk1, (0,))
    rv = lax.rev(tv1, (0,))
    m1 = ck > rk
    hk = jnp.where(m1, ck, rk)
    hv = jnp.where(m1, cv, rv)
    hk, hv = plsc.sort_key_val(hk, hv, descending=True)
    rhk = lax.rev(hk, (0,))
    rhv = lax.rev(hv, (0,))
    m2 = tk0 >= rhk
    ak = jnp.where(m2, tk0, rhk)
    av = jnp.where(m2, tv0, rhv)
    bk = jnp.where(m2, rhk, tk0)
    bv = jnp.where(m2, rhv, tv0)
    nk0, nv0 = plsc.sort_key_val(ak, av, descending=True)
    nk1, nv1 = plsc.sort_key_val(bk, bv, descending=True)
    return nk0, nv0, nk1, nv1


def _topk_kernel_body(ca_hbm, lab_hbm, out_hbm, buf, idxbuf, labbuf, outbuf,
                      gsem):
    c = lax.axis_index("c")
    s = lax.axis_index("s")
    row = s * 2 + c
    slab0 = pl.multiple_of((row // 8) * 8, 8)
    rloc = row % 8

    lane = lax.iota(jnp.int32, 16)
    t0k = jnp.full((16,), NEG, jnp.float32)
    t0v = jnp.zeros((16,), jnp.int32)
    t1k = jnp.full((16,), NEG, jnp.float32)
    t1v = jnp.zeros((16,), jnp.int32)
    thresh = jnp.float32(NEG)

    def chunk_body(k, carry):
        t0k, t0v, t1k, t1v, thresh = carry
        pltpu.sync_copy(
            ca_hbm.at[pl.ds(slab0, 8), pl.ds(k * SCCHUNK, SCCHUNK)], buf)
        base0 = k * SCCHUNK

        def vec_body(i, carry):
            t0k, t0v, t1k, t1v, thresh = carry
            v = buf[rloc, pl.ds(i * 16, 16)]
            m = v > thresh

            def do_merge(t0k, t0v, t1k, t1v):
                ck = jnp.where(m, v, NEG)
                cv = base0 + i * 16 + lane
                t0k, t0v, t1k, t1v = _merge(t0k, t0v, t1k, t1v, ck, cv)
                return t0k, t0v, t1k, t1v, jnp.min(t1k)

            def no_merge(t0k, t0v, t1k, t1v):
                return t0k, t0v, t1k, t1v, thresh

            return lax.cond(jnp.any(m), do_merge, no_merge, t0k, t0v, t1k, t1v)

        return lax.fori_loop(0, VPC, vec_body, (t0k, t0v, t1k, t1v, thresh))

    t0k, t0v, t1k, t1v, thresh = lax.fori_loop(
        0, 8, chunk_body, (t0k, t0v, t1k, t1v, thresh))

    idxbuf[pl.ds(0, 16)] = t0v
    idxbuf[pl.ds(16, 16)] = t1v
    pltpu.async_copy(lab_hbm.at[idxbuf], labbuf, gsem).wait()
    l0 = labbuf[pl.ds(0, 16)]
    l1 = labbuf[pl.ds(16, 16)]
    kth = jnp.min(t1k)
    denom = jnp.sum(t0k) + jnp.sum(t1k)
    sumlab = jnp.sum(l0) + jnp.sum(l1)
    dot = jnp.sum(t0k * l0) + jnp.sum(t1k * l1)
    recipv = 1.0 / (jnp.broadcast_to(denom, (16,)) + 1e-10)
    predv = jnp.broadcast_to(dot, (16,)) * recipv
    outv = jnp.where(lane == 0, jnp.broadcast_to(kth, (16,)),
           jnp.where(lane == 1, jnp.broadcast_to(sumlab * (1.0 / 32.0), (16,)),
           jnp.where(lane == 2, predv,
           jnp.where(lane == 3, recipv, 0.0))))
    outbuf[...] = outv
    pltpu.sync_copy(outbuf, out_hbm.at[pl.ds(row * 16, 16)])


def _topk_stats(ca2d, labels):
    mesh = plsc.VectorSubcoreMesh(core_axis_name="c", subcore_axis_name="s")
    f = pl.kernel(
        _topk_kernel_body,
        out_type=jax.ShapeDtypeStruct((B * 16,), jnp.float32),
        mesh=mesh,
        scratch_types=[
            pltpu.VMEM((8, SCCHUNK), jnp.float32),
            pltpu.VMEM((2 * 16,), jnp.int32),
            pltpu.VMEM((2 * 16,), jnp.float32),
            pltpu.VMEM((16,), jnp.float32),
            pltpu.SemaphoreType.DMA,
        ],
        compiler_params=pltpu.CompilerParams(needs_layout_passes=False),
    )
    return f(ca2d, labels).reshape(B, 16)


def _sm_body(ca_ref, thr_ref, rec_ref, sm_ref):
    ca = ca_ref[...]
    thr = thr_ref[...]
    rec = rec_ref[...]
    sm_ref[...] = jnp.where(ca >= thr, ca, 0.0) * rec


def _sm(ca2d, thr, rec):
    grid = NP // CBLK
    return pl.pallas_call(
        _sm_body,
        grid=(grid,),
        in_specs=[
            pl.BlockSpec((B, CBLK), lambda i: (0, i)),
            pl.BlockSpec((B, 1), lambda i: (0, 0)),
            pl.BlockSpec((B, 1), lambda i: (0, 0)),
        ],
        out_specs=pl.BlockSpec((B, CBLK), lambda i: (0, i)),
        out_shape=jax.ShapeDtypeStruct((B, N), jnp.float32),
    )(ca2d, thr, rec)


def kernel(query, cases, case_labels):
    qpat = jnp.tile(query, (1, GROUPS))
    cases3 = cases.reshape(NR, LANES)
    casesT = jnp.pad(cases.T, ((0, 0), (0, NP - N)))
    fa3, ca2d = _fa_ca(qpat, cases3, casesT)
    fa = fa3.reshape(B, N, F)

    stats = _topk_stats(ca2d, case_labels)
    kth = stats[:, 0:1]
    output = stats[:, 1]
    predicted = stats[:, 2]
    recip = stats[:, 3:4]

    sm = _sm(ca2d, kth, recip)
    return (fa, sm, output, predicted)


# speedup vs baseline: 2.1493x; 1.0002x over previous
import functools

import jax
import jax.numpy as jnp
from jax import lax
from jax.experimental import pallas as pl
from jax.experimental.pallas import tpu as pltpu
from jax.experimental.pallas import tpu_sc as plsc

B = 32
N = 100000
NP = 100352
F = 16
TOPK = 32
LANES = 128
GROUPS = LANES // F
NR = N // GROUPS
RB = 64
CBLK = 2048

SCCHUNK = NP // 8
VPC = SCCHUNK // 16
NEG = float("-inf")


def _fa_ca_body(qpat_ref, cases_ref, ct_ref, fa_ref, ca_ref):
    qp = qpat_ref[...]
    cs = cases_ref[...]
    fa = jnp.exp(-jnp.abs(qp[:, None, :] - cs[None, :, :]))
    fa_ref[...] = fa
    acc = jnp.zeros((B, RB * GROUPS), jnp.float32)
    for f in range(F):
        qf = qpat_ref[:, f:f + 1]
        cf = ct_ref[f:f + 1, :]
        acc = acc + jnp.exp(-jnp.abs(qf - cf))
    ca = acc * (1.0 / F)
    i = pl.program_id(0)
    gcol = i * (RB * GROUPS) + lax.broadcasted_iota(
        jnp.int32, (B, RB * GROUPS), 1)
    ca_ref[...] = jnp.where(gcol < N, ca, 0.0)


def _fa_ca(qpat, cases3, casesT):
    grid = (NR + RB - 1) // RB
    return pl.pallas_call(
        _fa_ca_body,
        grid=(grid,),
        in_specs=[
            pl.BlockSpec((B, LANES), lambda i: (0, 0)),
            pl.BlockSpec((RB, LANES), lambda i: (i, 0)),
            pl.BlockSpec((F, RB * GROUPS), lambda i: (0, i)),
        ],
        out_specs=[
            pl.BlockSpec((B, RB, LANES), lambda i: (0, i, 0)),
            pl.BlockSpec((B, RB * GROUPS), lambda i: (0, i)),
        ],
        out_shape=[
            jax.ShapeDtypeStruct((B, NR, LANES), jnp.float32),
            jax.ShapeDtypeStruct((B, NP), jnp.float32),
        ],
    )(qpat, cases3, casesT)


def _merge(tk0, tv0, tk1, tv1, ck, cv):
    ck, cv = plsc.sort_key_val(ck, cv, descending=True)
    rk = lax.rev(tk1, (0,))
    rv = lax.rev(tv1, (0,))
    m1 = ck > rk
    hk = jnp.where(m1, ck, rk)
    hv = jnp.where(m1, cv, rv)
    hk, hv = plsc.sort_key_val(hk, hv, descending=True)
    rhk = lax.rev(hk, (0,))
    rhv = lax.rev(hv, (0,))
    m2 = tk0 >= rhk
    ak = jnp.where(m2, tk0, rhk)
    av = jnp.where(m2, tv0, rhv)
    bk = jnp.where(m2, rhk, tk0)
    bv = jnp.where(m2, rhv, tv0)
    nk0, nv0 = plsc.sort_key_val(ak, av, descending=True)
    nk1, nv1 = plsc.sort_key_val(bk, bv, descending=True)
    return nk0, nv0, nk1, nv1


def _topk_kernel_body(ca_hbm, lab_hbm, out_hbm, buf, idxbuf, labbuf, outbuf,
                      gsem):
    c = lax.axis_index("c")
    s = lax.axis_index("s")
    row = s * 2 + c
    slab0 = pl.multiple_of((row // 8) * 8, 8)
    rloc = row % 8

    lane = lax.iota(jnp.int32, 16)
    t0k = jnp.full((16,), NEG, jnp.float32)
    t0v = jnp.zeros((16,), jnp.int32)
    t1k = jnp.full((16,), NEG, jnp.float32)
    t1v = jnp.zeros((16,), jnp.int32)
    thresh = jnp.float32(NEG)

    def chunk_body(k, carry):
        t0k, t0v, t1k, t1v, thresh = carry
        pltpu.sync_copy(
            ca_hbm.at[pl.ds(slab0, 8), pl.ds(k * SCCHUNK, SCCHUNK)], buf)
        base0 = k * SCCHUNK

        def vec_body(i, carry):
            t0k, t0v, t1k, t1v, thresh = carry
            v = buf[rloc, pl.ds(i * 16, 16)]
            m = v > thresh

            def do_merge(t0k, t0v, t1k, t1v):
                ck = jnp.where(m, v, NEG)
                cv = base0 + i * 16 + lane
                t0k, t0v, t1k, t1v = _merge(t0k, t0v, t1k, t1v, ck, cv)
                return t0k, t0v, t1k, t1v, jnp.min(t1k)

            def no_merge(t0k, t0v, t1k, t1v):
                return t0k, t0v, t1k, t1v, thresh

            return lax.cond(jnp.any(m), do_merge, no_merge, t0k, t0v, t1k, t1v)

        return lax.fori_loop(0, VPC, vec_body, (t0k, t0v, t1k, t1v, thresh))

    t0k, t0v, t1k, t1v, thresh = lax.fori_loop(
        0, 8, chunk_body, (t0k, t0v, t1k, t1v, thresh))

    idxbuf[pl.ds(0, 16)] = t0v
    idxbuf[pl.ds(16, 16)] = t1v
    pltpu.async_copy(lab_hbm.at[idxbuf], labbuf, gsem).wait()
    l0 = labbuf[pl.ds(0, 16)]
    l1 = labbuf[pl.ds(16, 16)]
    kth = jnp.min(t1k)
    denom = jnp.sum(t0k) + jnp.sum(t1k)
    sumlab = jnp.sum(l0) + jnp.sum(l1)
    dot = jnp.sum(t0k * l0) + jnp.sum(t1k * l1)
    recipv = 1.0 / (jnp.broadcast_to(denom, (16,)) + 1e-10)
    predv = jnp.broadcast_to(dot, (16,)) * recipv
    outv = jnp.where(lane == 0, jnp.broadcast_to(kth, (16,)),
           jnp.where(lane == 1, jnp.broadcast_to(sumlab * (1.0 / 32.0), (16,)),
           jnp.where(lane == 2, predv,
           jnp.where(lane == 3, recipv, 0.0))))
    outbuf[...] = outv
    pltpu.sync_copy(outbuf, out_hbm.at[pl.ds(row * 16, 16)])


def _topk_stats(ca2d, labels):
    mesh = plsc.VectorSubcoreMesh(core_axis_name="c", subcore_axis_name="s")
    f = pl.kernel(
        _topk_kernel_body,
        out_type=jax.ShapeDtypeStruct((B * 16,), jnp.float32),
        mesh=mesh,
        scratch_types=[
            pltpu.VMEM((8, SCCHUNK), jnp.float32),
            pltpu.VMEM((2 * 16,), jnp.int32),
            pltpu.VMEM((2 * 16,), jnp.float32),
            pltpu.VMEM((16,), jnp.float32),
            pltpu.SemaphoreType.DMA,
        ],
        compiler_params=pltpu.CompilerParams(needs_layout_passes=False),
    )
    return f(ca2d, labels).reshape(B, 16)


def _sm_body(ca_ref, thr_ref, rec_ref, sm_ref):
    ca = ca_ref[...]
    thr = thr_ref[...]
    rec = rec_ref[...]
    sm_ref[...] = jnp.where(ca >= thr, ca, 0.0) * rec


def _sm(ca2d, thr, rec):
    grid = NP // CBLK
    return pl.pallas_call(
        _sm_body,
        grid=(grid,),
        in_specs=[
            pl.BlockSpec((B, CBLK), lambda i: (0, i)),
            pl.BlockSpec((B, 1), lambda i: (0, 0)),
            pl.BlockSpec((B, 1), lambda i: (0, 0)),
        ],
        out_specs=pl.BlockSpec((B, CBLK), lambda i: (0, i)),
        out_shape=jax.ShapeDtypeStruct((B, N), jnp.float32),
    )(ca2d, thr, rec)


def kernel(query, cases, case_labels):
    qpat = jnp.tile(query, (1, GROUPS))
    cases3 = cases.reshape(NR, LANES)
    casesT = lax.dot_general(jnp.eye(F, dtype=jnp.float32), cases,
                             (((1,), (1,)), ((), ())),
                             precision=lax.Precision.HIGHEST)
    fa3, ca2d = _fa_ca(qpat, cases3, casesT)
    fa = fa3.reshape(B, N, F)

    stats = _topk_stats(ca2d, case_labels)
    kth = stats[:, 0:1]
    output = stats[:, 1]
    predicted = stats[:, 2]
    recip = stats[:, 3:4]

    sm = _sm(ca2d, kth, recip)
    return (fa, sm, output, predicted)


# speedup vs baseline: 5.5054x; 2.5615x over previous
import functools

import jax
import jax.numpy as jnp
from jax import lax
from jax.experimental import pallas as pl
from jax.experimental.pallas import tpu as pltpu
from jax.experimental.pallas import tpu_sc as plsc

B = 32
N = 100000
NP = 100352
F = 16
TOPK = 32
LANES = 128
GROUPS = LANES // F
NR = N // GROUPS
RB = 64
CBLK = 2048

SCCHUNK = NP // 8
VPC = SCCHUNK // 16
NEG = float("-inf")


WBLK = 2048


def _fa_ca_body(q_ref, ct_ref, fa_ref, ca_ref):
    acc = jnp.zeros((B, WBLK), jnp.float32)
    for f in range(F):
        qf = q_ref[:, f:f + 1]
        cf = ct_ref[f:f + 1, :]
        slab = jnp.exp(-jnp.abs(qf - cf))
        fa_ref[:, f:f + 1, :] = slab[:, None, :]
        acc = acc + slab
    ca = acc * (1.0 / F)
    i = pl.program_id(0)
    gcol = i * WBLK + lax.broadcasted_iota(jnp.int32, (B, WBLK), 1)
    ca_ref[...] = jnp.where(gcol < N, ca, 0.0)


def _fa_ca(query, casesT):
    grid = NP // WBLK
    return pl.pallas_call(
        _fa_ca_body,
        grid=(grid,),
        in_specs=[
            pl.BlockSpec((B, F), lambda i: (0, 0)),
            pl.BlockSpec((F, WBLK), lambda i: (0, i)),
        ],
        out_specs=[
            pl.BlockSpec((B, F, WBLK), lambda i: (0, 0, i)),
            pl.BlockSpec((B, WBLK), lambda i: (0, i)),
        ],
        out_shape=[
            jax.ShapeDtypeStruct((B, F, N), jnp.float32),
            jax.ShapeDtypeStruct((B, NP), jnp.float32),
        ],
    )(query, casesT)


def _merge(tk0, tv0, tk1, tv1, ck, cv):
    ck, cv = plsc.sort_key_val(ck, cv, descending=True)
    rk = lax.rev(tk1, (0,))
    rv = lax.rev(tv1, (0,))
    m1 = ck > rk
    hk = jnp.where(m1, ck, rk)
    hv = jnp.where(m1, cv, rv)
    hk, hv = plsc.sort_key_val(hk, hv, descending=True)
    rhk = lax.rev(hk, (0,))
    rhv = lax.rev(hv, (0,))
    m2 = tk0 >= rhk
    ak = jnp.where(m2, tk0, rhk)
    av = jnp.where(m2, tv0, rhv)
    bk = jnp.where(m2, rhk, tk0)
    bv = jnp.where(m2, rhv, tv0)
    nk0, nv0 = plsc.sort_key_val(ak, av, descending=True)
    nk1, nv1 = plsc.sort_key_val(bk, bv, descending=True)
    return nk0, nv0, nk1, nv1


def _topk_kernel_body(ca_hbm, lab_hbm, out_hbm, buf, idxbuf, labbuf, outbuf,
                      gsem):
    c = lax.axis_index("c")
    s = lax.axis_index("s")
    row = s * 2 + c
    slab0 = pl.multiple_of((row // 8) * 8, 8)
    rloc = row % 8

    lane = lax.iota(jnp.int32, 16)
    t0k = jnp.full((16,), NEG, jnp.float32)
    t0v = jnp.zeros((16,), jnp.int32)
    t1k = jnp.full((16,), NEG, jnp.float32)
    t1v = jnp.zeros((16,), jnp.int32)
    thresh = jnp.float32(NEG)

    def chunk_body(k, carry):
        t0k, t0v, t1k, t1v, thresh = carry
        pltpu.sync_copy(
            ca_hbm.at[pl.ds(slab0, 8), pl.ds(k * SCCHUNK, SCCHUNK)], buf)
        base0 = k * SCCHUNK

        def vec_body(i, carry):
            t0k, t0v, t1k, t1v, thresh = carry
            v = buf[rloc, pl.ds(i * 16, 16)]
            m = v > thresh

            def do_merge(t0k, t0v, t1k, t1v):
                ck = jnp.where(m, v, NEG)
                cv = base0 + i * 16 + lane
                t0k, t0v, t1k, t1v = _merge(t0k, t0v, t1k, t1v, ck, cv)
                return t0k, t0v, t1k, t1v, jnp.min(t1k)

            def no_merge(t0k, t0v, t1k, t1v):
                return t0k, t0v, t1k, t1v, thresh

            return lax.cond(jnp.any(m), do_merge, no_merge, t0k, t0v, t1k, t1v)

        return lax.fori_loop(0, VPC, vec_body, (t0k, t0v, t1k, t1v, thresh))

    t0k, t0v, t1k, t1v, thresh = lax.fori_loop(
        0, 8, chunk_body, (t0k, t0v, t1k, t1v, thresh))

    idxbuf[pl.ds(0, 16)] = t0v
    idxbuf[pl.ds(16, 16)] = t1v
    pltpu.async_copy(lab_hbm.at[idxbuf], labbuf, gsem).wait()
    l0 = labbuf[pl.ds(0, 16)]
    l1 = labbuf[pl.ds(16, 16)]
    kth = jnp.min(t1k)
    denom = jnp.sum(t0k) + jnp.sum(t1k)
    sumlab = jnp.sum(l0) + jnp.sum(l1)
    dot = jnp.sum(t0k * l0) + jnp.sum(t1k * l1)
    recipv = 1.0 / (jnp.broadcast_to(denom, (16,)) + 1e-10)
    predv = jnp.broadcast_to(dot, (16,)) * recipv
    outv = jnp.where(lane == 0, jnp.broadcast_to(kth, (16,)),
           jnp.where(lane == 1, jnp.broadcast_to(sumlab * (1.0 / 32.0), (16,)),
           jnp.where(lane == 2, predv,
           jnp.where(lane == 3, recipv, 0.0))))
    outbuf[...] = outv
    pltpu.sync_copy(outbuf, out_hbm.at[pl.ds(row * 16, 16)])


def _topk_stats(ca2d, labels):
    mesh = plsc.VectorSubcoreMesh(core_axis_name="c", subcore_axis_name="s")
    f = pl.kernel(
        _topk_kernel_body,
        out_type=jax.ShapeDtypeStruct((B * 16,), jnp.float32),
        mesh=mesh,
        scratch_types=[
            pltpu.VMEM((8, SCCHUNK), jnp.float32),
            pltpu.VMEM((2 * 16,), jnp.int32),
            pltpu.VMEM((2 * 16,), jnp.float32),
            pltpu.VMEM((16,), jnp.float32),
            pltpu.SemaphoreType.DMA,
        ],
        compiler_params=pltpu.CompilerParams(needs_layout_passes=False),
    )
    return f(ca2d, labels).reshape(B, 16)


def _sm_body(ca_ref, thr_ref, rec_ref, sm_ref):
    ca = ca_ref[...]
    thr = thr_ref[...]
    rec = rec_ref[...]
    sm_ref[...] = jnp.where(ca >= thr, ca, 0.0) * rec


def _sm(ca2d, thr, rec):
    grid = NP // CBLK
    return pl.pallas_call(
        _sm_body,
        grid=(grid,),
        in_specs=[
            pl.BlockSpec((B, CBLK), lambda i: (0, i)),
            pl.BlockSpec((B, 1), lambda i: (0, 0)),
            pl.BlockSpec((B, 1), lambda i: (0, 0)),
        ],
        out_specs=pl.BlockSpec((B, CBLK), lambda i: (0, i)),
        out_shape=jax.ShapeDtypeStruct((B, N), jnp.float32),
    )(ca2d, thr, rec)


def kernel(query, cases, case_labels):
    casesT = lax.dot_general(jnp.eye(F, dtype=jnp.float32), cases,
                             (((1,), (1,)), ((), ())),
                             precision=lax.Precision.HIGHEST)
    faT, ca2d = _fa_ca(query, casesT)
    fa = faT.transpose(0, 2, 1)

    stats = _topk_stats(ca2d, case_labels)
    kth = stats[:, 0:1]
    output = stats[:, 1]
    predicted = stats[:, 2]
    recip = stats[:, 3:4]

    sm = _sm(ca2d, kth, recip)
    return (fa, sm, output, predicted)


# speedup vs baseline: 11.6662x; 2.1190x over previous
import functools

import jax
import jax.numpy as jnp
from jax import lax
from jax.experimental import pallas as pl
from jax.experimental.pallas import tpu as pltpu
from jax.experimental.pallas import tpu_sc as plsc

B = 32
N = 100000
NP = 100352
F = 16
TOPK = 32
LANES = 128
GROUPS = LANES // F
NR = N // GROUPS
RB = 64
CBLK = 2048

SCCHUNK = NP // 8
VPC = SCCHUNK // 16
UNROLL = 8
NEG = float("-inf")


WBLK = 2048


def _fa_ca_body(q_ref, ct_ref, fa_ref, ca_ref):
    acc = jnp.zeros((B, WBLK), jnp.float32)
    for f in range(F):
        qf = q_ref[:, f:f + 1]
        cf = ct_ref[f:f + 1, :]
        slab = jnp.exp(-jnp.abs(qf - cf))
        fa_ref[:, f:f + 1, :] = slab[:, None, :]
        acc = acc + slab
    ca = acc * (1.0 / F)
    i = pl.program_id(0)
    gcol = i * WBLK + lax.broadcasted_iota(jnp.int32, (B, WBLK), 1)
    ca_ref[...] = jnp.where(gcol < N, ca, 0.0)


def _fa_ca(query, casesT):
    grid = NP // WBLK
    return pl.pallas_call(
        _fa_ca_body,
        grid=(grid,),
        in_specs=[
            pl.BlockSpec((B, F), lambda i: (0, 0)),
            pl.BlockSpec((F, WBLK), lambda i: (0, i)),
        ],
        out_specs=[
            pl.BlockSpec((B, F, WBLK), lambda i: (0, 0, i)),
            pl.BlockSpec((B, WBLK), lambda i: (0, i)),
        ],
        out_shape=[
            jax.ShapeDtypeStruct((B, F, N), jnp.float32),
            jax.ShapeDtypeStruct((B, NP), jnp.float32),
        ],
    )(query, casesT)


def _merge(tk0, tv0, tk1, tv1, ck, cv):
    ck, cv = plsc.sort_key_val(ck, cv, descending=True)
    rk = lax.rev(tk1, (0,))
    rv = lax.rev(tv1, (0,))
    m1 = ck > rk
    hk = jnp.where(m1, ck, rk)
    hv = jnp.where(m1, cv, rv)
    hk, hv = plsc.sort_key_val(hk, hv, descending=True)
    rhk = lax.rev(hk, (0,))
    rhv = lax.rev(hv, (0,))
    m2 = tk0 >= rhk
    ak = jnp.where(m2, tk0, rhk)
    av = jnp.where(m2, tv0, rhv)
    bk = jnp.where(m2, rhk, tk0)
    bv = jnp.where(m2, rhv, tv0)
    nk0, nv0 = plsc.sort_key_val(ak, av, descending=True)
    nk1, nv1 = plsc.sort_key_val(bk, bv, descending=True)
    return nk0, nv0, nk1, nv1


def _topk_kernel_body(ca_hbm, lab_hbm, out_hbm, buf, idxbuf, labbuf, outbuf,
                      gsem):
    c = lax.axis_index("c")
    s = lax.axis_index("s")
    row = s * 2 + c
    slab0 = pl.multiple_of((row // 8) * 8, 8)
    rloc = row % 8

    lane = lax.iota(jnp.int32, 16)
    t0k = jnp.full((16,), NEG, jnp.float32)
    t0v = jnp.zeros((16,), jnp.int32)
    t1k = jnp.full((16,), NEG, jnp.float32)
    t1v = jnp.zeros((16,), jnp.int32)
    thresh = jnp.float32(NEG)

    def chunk_body(k, carry):
        t0k, t0v, t1k, t1v, thresh = carry
        pltpu.sync_copy(
            ca_hbm.at[pl.ds(slab0, 8), pl.ds(k * SCCHUNK, SCCHUNK)], buf)
        base0 = k * SCCHUNK

        def vec_group(j, carry):
            t0k, t0v, t1k, t1v, thresh = carry
            vs = [buf[rloc, pl.ds((j * UNROLL + u) * 16, 16)]
                  for u in range(UNROLL)]
            gmax = vs[0]
            for u in range(1, UNROLL):
                gmax = jnp.maximum(gmax, vs[u])

            def slow(t0k, t0v, t1k, t1v, thresh):
                for u in range(UNROLL):
                    v = vs[u]
                    m = v > thresh
                    ck = jnp.where(m, v, NEG)
                    cv = base0 + (j * UNROLL + u) * 16 + lane
                    t0k, t0v, t1k, t1v = _merge(t0k, t0v, t1k, t1v, ck, cv)
                    thresh = jnp.min(t1k)
                return t0k, t0v, t1k, t1v, thresh

            def fast(t0k, t0v, t1k, t1v, thresh):
                return t0k, t0v, t1k, t1v, thresh

            return lax.cond(jnp.any(gmax > thresh), slow, fast,
                            t0k, t0v, t1k, t1v, thresh)

        return lax.fori_loop(0, VPC // UNROLL, vec_group,
                             (t0k, t0v, t1k, t1v, thresh))

    t0k, t0v, t1k, t1v, thresh = lax.fori_loop(
        0, 8, chunk_body, (t0k, t0v, t1k, t1v, thresh))

    idxbuf[pl.ds(0, 16)] = t0v
    idxbuf[pl.ds(16, 16)] = t1v
    pltpu.async_copy(lab_hbm.at[idxbuf], labbuf, gsem).wait()
    l0 = labbuf[pl.ds(0, 16)]
    l1 = labbuf[pl.ds(16, 16)]
    kth = jnp.min(t1k)
    denom = jnp.sum(t0k) + jnp.sum(t1k)
    sumlab = jnp.sum(l0) + jnp.sum(l1)
    dot = jnp.sum(t0k * l0) + jnp.sum(t1k * l1)
    recipv = 1.0 / (jnp.broadcast_to(denom, (16,)) + 1e-10)
    predv = jnp.broadcast_to(dot, (16,)) * recipv
    outv = jnp.where(lane == 0, jnp.broadcast_to(kth, (16,)),
           jnp.where(lane == 1, jnp.broadcast_to(sumlab * (1.0 / 32.0), (16,)),
           jnp.where(lane == 2, predv,
           jnp.where(lane == 3, recipv, 0.0))))
    outbuf[...] = outv
    pltpu.sync_copy(outbuf, out_hbm.at[pl.ds(row * 16, 16)])


def _topk_stats(ca2d, labels):
    mesh = plsc.VectorSubcoreMesh(core_axis_name="c", subcore_axis_name="s")
    f = pl.kernel(
        _topk_kernel_body,
        out_type=jax.ShapeDtypeStruct((B * 16,), jnp.float32),
        mesh=mesh,
        scratch_types=[
            pltpu.VMEM((8, SCCHUNK), jnp.float32),
            pltpu.VMEM((2 * 16,), jnp.int32),
            pltpu.VMEM((2 * 16,), jnp.float32),
            pltpu.VMEM((16,), jnp.float32),
            pltpu.SemaphoreType.DMA,
        ],
        compiler_params=pltpu.CompilerParams(needs_layout_passes=False),
    )
    return f(ca2d, labels).reshape(B, 16)


def _sm_body(ca_ref, thr_ref, rec_ref, sm_ref):
    ca = ca_ref[...]
    thr = thr_ref[...]
    rec = rec_ref[...]
    sm_ref[...] = jnp.where(ca >= thr, ca, 0.0) * rec


def _sm(ca2d, thr, rec):
    grid = NP // CBLK
    return pl.pallas_call(
        _sm_body,
        grid=(grid,),
        in_specs=[
            pl.BlockSpec((B, CBLK), lambda i: (0, i)),
            pl.BlockSpec((B, 1), lambda i: (0, 0)),
            pl.BlockSpec((B, 1), lambda i: (0, 0)),
        ],
        out_specs=pl.BlockSpec((B, CBLK), lambda i: (0, i)),
        out_shape=jax.ShapeDtypeStruct((B, N), jnp.float32),
    )(ca2d, thr, rec)


def kernel(query, cases, case_labels):
    casesT = lax.dot_general(jnp.eye(F, dtype=jnp.float32), cases,
                             (((1,), (1,)), ((), ())),
                             precision=lax.Precision.HIGHEST)
    faT, ca2d = _fa_ca(query, casesT)
    fa = faT.transpose(0, 2, 1)

    stats = _topk_stats(ca2d, case_labels)
    kth = stats[:, 0:1]
    output = stats[:, 1]
    predicted = stats[:, 2]
    recip = stats[:, 3:4]

    sm = _sm(ca2d, kth, recip)
    return (fa, sm, output, predicted)
